# baseline (speedup 1.0000x reference)
import functools
import jax
import jax.numpy as jnp
from jax import lax
from jax.experimental import pallas as pl
from jax.experimental.pallas import tpu as pltpu
from jax.experimental.pallas import tpu_sc as plsc

HIDDEN = 1024
INTER = 2048
NUM_EXPERTS = 8
T = 2048
BT = 256
CAP = T * 2 + NUM_EXPERTS * BT
NB = CAP // BT
NTILES = 32
SLOTS_PER_TILE = CAP // NTILES
TOK_PER_TILE = T // NTILES


def _router_body(x_ref, gate_ref, pos0_ref, pos1_ref, wts0_ref, wts1_ref,
                 meta_ref):
    x = x_ref[...]
    g = gate_ref[...]
    logits = lax.dot_general(x, g, (((1,), (1,)), ((), ())),
                             preferred_element_type=jnp.float32)
    p = jax.nn.softmax(logits, axis=1)
    eidx = lax.broadcasted_iota(jnp.int32, p.shape, 1)
    m1 = jnp.max(p, axis=1, keepdims=True)
    e0 = jnp.min(jnp.where(p == m1, eidx, NUM_EXPERTS), axis=1, keepdims=True)
    a0 = eidx == e0
    p2 = jnp.where(a0, -1.0, p)
    m2 = jnp.max(p2, axis=1, keepdims=True)
    e1 = jnp.min(jnp.where(p2 == m2, eidx, NUM_EXPERTS), axis=1, keepdims=True)
    a1 = eidx == e1
    denom = m1 + m2
    wts0_ref[...] = (m1 / denom).reshape(T)
    wts1_ref[...] = (m2 / denom).reshape(T)

    s = jnp.where(a0 | a1, 1.0, 0.0)
    r2 = lax.broadcasted_iota(jnp.int32, (T, T), 0)
    c2 = lax.broadcasted_iota(jnp.int32, (T, T), 1)
    ltri = jnp.where(c2 <= r2, 1.0, 0.0)
    cinc = lax.dot_general(ltri, s, (((1,), (0,)), ((), ())),
                           preferred_element_type=jnp.float32)
    cexc = cinc - s
    counts = cinc[T - 1:T, :]
    cnt_i = counts.astype(jnp.int32)
    pc_i = ((cnt_i + (BT - 1)) // BT) * BT
    pc_f = pc_i.astype(jnp.float32)
    er = lax.broadcasted_iota(jnp.int32, (NUM_EXPERTS, NUM_EXPERTS), 0)
    ec = lax.broadcasted_iota(jnp.int32, (NUM_EXPERTS, NUM_EXPERTS), 1)
    sut = jnp.where(er < ec, 1.0, 0.0)
    start_f = lax.dot_general(pc_f, sut, (((1,), (0,)), ((), ())),
                              preferred_element_type=jnp.float32)

    posmat = cexc + start_f
    pos0 = jnp.sum(jnp.where(a0, posmat, 0.0), axis=1, keepdims=True)
    pos1 = jnp.sum(jnp.where(a1, posmat, 0.0), axis=1, keepdims=True)
    pos0_ref[...] = pos0.astype(jnp.int32).reshape(T)
    pos1_ref[...] = pos1.astype(jnp.int32).reshape(T)

    bb = lax.broadcasted_iota(jnp.int32, (1, 128), 1)
    slot0 = (bb * BT).astype(jnp.float32)
    total_pad = jnp.sum(pc_f, axis=1, keepdims=True)
    nb_act = total_pad / float(BT)
    be = jnp.zeros((1, 128), jnp.float32)
    rel = jnp.zeros((1, 128), jnp.float32)
    cnt_at = jnp.zeros((1, 128), jnp.float32)
    for e in range(NUM_EXPERTS):
        st_e = start_f[0:1, e:e + 1]
        pc_e = pc_f[0:1, e:e + 1]
        ct_e = counts[0:1, e:e + 1]
        in_e = jnp.where((slot0 >= st_e) & (slot0 < st_e + pc_e), 1.0, 0.0)
        be = be + in_e * float(e)
        rel = rel + in_e * (slot0 - st_e)
        cnt_at = cnt_at + in_e * ct_e
    eids = lax.broadcasted_iota(jnp.int32, (1, NUM_EXPERTS), 1).astype(jnp.float32)
    last_e = jnp.max(jnp.where(pc_f > 0, eids, 0.0), axis=1, keepdims=True)
    in_range = bb.astype(jnp.float32) < nb_act
    be_full = jnp.where(in_range, be, last_e)
    act = jnp.where(in_range & (rel < cnt_at), 1.0, 0.0)
    meta_ref[...] = jnp.concatenate([be_full, act], axis=0).astype(jnp.int32)


def _dispatch_body(x_hbm, pos0_hbm, pos1_hbm, xs_hbm,
                   idx0_v, idx1_v, rows_v, sem):
    wid = lax.axis_index("s") * 2 + lax.axis_index("c")
    tbase = wid * TOK_PER_TILE
    for c in range(TOK_PER_TILE // 32):
        tb = tbase + 32 * c
        pltpu.sync_copy(pos0_hbm.at[pl.ds(tb, 32)], idx0_v)
        pltpu.sync_copy(pos1_hbm.at[pl.ds(tb, 32)], idx1_v)
        pltpu.sync_copy(x_hbm.at[pl.ds(tb, 32)], rows_v)
        pltpu.sync_copy(rows_v, xs_hbm.at[idx0_v])
        pltpu.sync_copy(rows_v, xs_hbm.at[idx1_v])


def _make_dispatch():
    mesh = plsc.VectorSubcoreMesh(core_axis_name="c", subcore_axis_name="s")
    return functools.partial(
        pl.kernel, _dispatch_body, mesh=mesh,
        out_type=jax.ShapeDtypeStruct((CAP, HIDDEN), jnp.float32),
        scratch_types=[
            pltpu.VMEM((32,), jnp.int32),
            pltpu.VMEM((32,), jnp.int32),
            pltpu.VMEM((32, HIDDEN), jnp.float32),
            pltpu.SemaphoreType.DMA,
        ],
    )()


BI = INTER // 2


def _ffn_body(meta_ref, xs_ref, w1_ref, w2_ref, w3_ref, ys_ref):
    b = pl.program_id(0)
    ic = pl.program_id(1)

    @pl.when(meta_ref[1, b] > 0)
    def _():
        x = xs_ref[...]
        w1 = w1_ref[0]
        w3 = w3_ref[0]
        w2 = w2_ref[0]
        h = lax.dot_general(x, w1, (((1,), (1,)), ((), ())),
                            preferred_element_type=jnp.float32)
        u = lax.dot_general(x, w3, (((1,), (1,)), ((), ())),
                            preferred_element_type=jnp.float32)
        gg = h * jax.nn.sigmoid(h) * u
        o = lax.dot_general(gg, w2, (((1,), (1,)), ((), ())),
                            preferred_element_type=jnp.float32)

        @pl.when(ic == 0)
        def _():
            ys_ref[...] = o

        @pl.when(ic == 1)
        def _():
            ys_ref[...] += o


def _combine_body(ys_hbm, pos0_hbm, pos1_hbm, wts0_hbm, wts1_hbm, out_hbm,
                  idx0_v, idx1_v, w0_v, w1_v, bufa_v, bufb_v, outb_v,
                  sg0, sg1, sst):
    wid = lax.axis_index("s") * 2 + lax.axis_index("c")
    tb0 = wid * TOK_PER_TILE
    nch = TOK_PER_TILE // 16
    pltpu.sync_copy(pos0_hbm.at[pl.ds(wid * nch, nch)], idx0_v)
    pltpu.sync_copy(pos1_hbm.at[pl.ds(wid * nch, nch)], idx1_v)
    pltpu.sync_copy(wts0_hbm.at[pl.ds(tb0, TOK_PER_TILE)],
                    w0_v.at[pl.ds(0, TOK_PER_TILE)])
    pltpu.sync_copy(wts1_hbm.at[pl.ds(tb0, TOK_PER_TILE)],
                    w1_v.at[pl.ds(0, TOK_PER_TILE)])

    def issue(c):
        r = c % 2
        s = sg0 if r == 0 else sg1
        ca = pltpu.async_copy(ys_hbm.at[idx0_v.at[c]], bufa_v.at[r], s)
        cb = pltpu.async_copy(ys_hbm.at[idx1_v.at[c]], bufb_v.at[r], s)
        return ca, cb

    inflight = issue(0)
    store = None
    for c in range(nch):
        nxt = issue(c + 1) if c + 1 < nch else None
        inflight[0].wait()
        inflight[1].wait()
        inflight = nxt
        r = c % 2
        if store is not None:
            store.wait()

        def comb(j, carry):
            w0 = w0_v[pl.ds(c * 16 + j, 16)][0]
            w1 = w1_v[pl.ds(c * 16 + j, 16)][0]
            for cc in range(HIDDEN // 16):
                va = bufa_v[r, j, pl.ds(cc * 16, 16)]
                vb = bufb_v[r, j, pl.ds(cc * 16, 16)]
                outb_v[r, j, pl.ds(cc * 16, 16)] = w0 * va + w1 * vb
            return carry

        lax.fori_loop(0, 16, comb, 0)
        store = pltpu.async_copy(outb_v.at[r],
                                 out_hbm.at[pl.ds(tb0 + c * 16, 16)], sst)
    store.wait()


def _make_combine():
    mesh = plsc.VectorSubcoreMesh(core_axis_name="c", subcore_axis_name="s")
    return functools.partial(
        pl.kernel, _combine_body, mesh=mesh,
        out_type=jax.ShapeDtypeStruct((T, HIDDEN), jnp.float32),
        scratch_types=[
            pltpu.VMEM((TOK_PER_TILE // 16, 16), jnp.int32),
            pltpu.VMEM((TOK_PER_TILE // 16, 16), jnp.int32),
            pltpu.VMEM((TOK_PER_TILE + 16,), jnp.float32),
            pltpu.VMEM((TOK_PER_TILE + 16,), jnp.float32),
            pltpu.VMEM((2, 16, HIDDEN), jnp.float32),
            pltpu.VMEM((2, 16, HIDDEN), jnp.float32),
            pltpu.VMEM((2, 16, HIDDEN), jnp.float32),
            pltpu.SemaphoreType.DMA,
            pltpu.SemaphoreType.DMA,
            pltpu.SemaphoreType.DMA,
        ],
    )()


def _combine(ys, pos0, pos1, wts0, wts1):
    return _make_combine()(ys, pos0.reshape(T // 16, 16),
                           pos1.reshape(T // 16, 16), wts0, wts1)


def kernel(hidden_states, gate_w, w1, w2, w3):
    b, s, h = hidden_states.shape
    x = hidden_states.reshape(-1, h)

    pos0, pos1, wts0, wts1, meta = pl.pallas_call(
        _router_body,
        out_shape=[
            jax.ShapeDtypeStruct((T,), jnp.int32),
            jax.ShapeDtypeStruct((T,), jnp.int32),
            jax.ShapeDtypeStruct((T,), jnp.float32),
            jax.ShapeDtypeStruct((T,), jnp.float32),
            jax.ShapeDtypeStruct((2, 128), jnp.int32),
        ],
    )(x, gate_w)

    xs = _make_dispatch()(x, pos0, pos1)

    ys = pl.pallas_call(
        _ffn_body,
        grid_spec=pltpu.PrefetchScalarGridSpec(
            num_scalar_prefetch=1,
            grid=(NB, 2),
            in_specs=[
                pl.BlockSpec((BT, HIDDEN), lambda i, ic, m: (i, 0)),
                pl.BlockSpec((1, BI, HIDDEN), lambda i, ic, m: (m[0, i], ic, 0)),
                pl.BlockSpec((1, HIDDEN, BI), lambda i, ic, m: (m[0, i], 0, ic)),
                pl.BlockSpec((1, BI, HIDDEN), lambda i, ic, m: (m[0, i], ic, 0)),
            ],
            out_specs=pl.BlockSpec((BT, HIDDEN), lambda i, ic, m: (i, 0)),
        ),
        out_shape=jax.ShapeDtypeStruct((CAP, HIDDEN), jnp.float32),
        compiler_params=pltpu.CompilerParams(
            vmem_limit_bytes=62 * 1024 * 1024),
    )(meta, xs, w1, w2, w3)

    out = _combine(ys, pos0, pos1, wts0, wts1)
    return out.reshape(b, s, h)


# speedup vs baseline: 1.3895x; 1.3895x over previous
import functools
import jax
import jax.numpy as jnp
from jax import lax
from jax.experimental import pallas as pl
from jax.experimental.pallas import tpu as pltpu
from jax.experimental.pallas import tpu_sc as plsc

HIDDEN = 1024
INTER = 2048
NUM_EXPERTS = 8
T = 2048
BT = 512
CAP = T * 2 + NUM_EXPERTS * BT
NB = CAP // BT
NTILES = 32
SLOTS_PER_TILE = CAP // NTILES
TOK_PER_TILE = T // NTILES


def _router_body(x_ref, gate_ref, pos0_ref, pos1_ref, wts0_ref, wts1_ref,
                 meta_ref):
    x = x_ref[...]
    g = gate_ref[...]
    logits = lax.dot_general(x, g, (((1,), (1,)), ((), ())),
                             preferred_element_type=jnp.float32)
    p = jax.nn.softmax(logits, axis=1)
    eidx = lax.broadcasted_iota(jnp.int32, p.shape, 1)
    m1 = jnp.max(p, axis=1, keepdims=True)
    e0 = jnp.min(jnp.where(p == m1, eidx, NUM_EXPERTS), axis=1, keepdims=True)
    a0 = eidx == e0
    p2 = jnp.where(a0, -1.0, p)
    m2 = jnp.max(p2, axis=1, keepdims=True)
    e1 = jnp.min(jnp.where(p2 == m2, eidx, NUM_EXPERTS), axis=1, keepdims=True)
    a1 = eidx == e1
    denom = m1 + m2
    wts0_ref[...] = (m1 / denom).reshape(T)
    wts1_ref[...] = (m2 / denom).reshape(T)

    s = jnp.where(a0 | a1, 1.0, 0.0)
    r2 = lax.broadcasted_iota(jnp.int32, (T, T), 0)
    c2 = lax.broadcasted_iota(jnp.int32, (T, T), 1)
    ltri = jnp.where(c2 <= r2, 1.0, 0.0)
    cinc = lax.dot_general(ltri, s, (((1,), (0,)), ((), ())),
                           preferred_element_type=jnp.float32)
    cexc = cinc - s
    counts = cinc[T - 1:T, :]
    cnt_i = counts.astype(jnp.int32)
    pc_i = ((cnt_i + (BT - 1)) // BT) * BT
    pc_f = pc_i.astype(jnp.float32)
    er = lax.broadcasted_iota(jnp.int32, (NUM_EXPERTS, NUM_EXPERTS), 0)
    ec = lax.broadcasted_iota(jnp.int32, (NUM_EXPERTS, NUM_EXPERTS), 1)
    sut = jnp.where(er < ec, 1.0, 0.0)
    start_f = lax.dot_general(pc_f, sut, (((1,), (0,)), ((), ())),
                              preferred_element_type=jnp.float32)

    posmat = cexc + start_f
    pos0 = jnp.sum(jnp.where(a0, posmat, 0.0), axis=1, keepdims=True)
    pos1 = jnp.sum(jnp.where(a1, posmat, 0.0), axis=1, keepdims=True)
    pos0_ref[...] = pos0.astype(jnp.int32).reshape(T)
    pos1_ref[...] = pos1.astype(jnp.int32).reshape(T)

    bb = lax.broadcasted_iota(jnp.int32, (1, 128), 1)
    slot0 = (bb * BT).astype(jnp.float32)
    total_pad = jnp.sum(pc_f, axis=1, keepdims=True)
    nb_act = total_pad / float(BT)
    be = jnp.zeros((1, 128), jnp.float32)
    rel = jnp.zeros((1, 128), jnp.float32)
    cnt_at = jnp.zeros((1, 128), jnp.float32)
    for e in range(NUM_EXPERTS):
        st_e = start_f[0:1, e:e + 1]
        pc_e = pc_f[0:1, e:e + 1]
        ct_e = counts[0:1, e:e + 1]
        in_e = jnp.where((slot0 >= st_e) & (slot0 < st_e + pc_e), 1.0, 0.0)
        be = be + in_e * float(e)
        rel = rel + in_e * (slot0 - st_e)
        cnt_at = cnt_at + in_e * ct_e
    eids = lax.broadcasted_iota(jnp.int32, (1, NUM_EXPERTS), 1).astype(jnp.float32)
    last_e = jnp.max(jnp.where(pc_f > 0, eids, 0.0), axis=1, keepdims=True)
    in_range = bb.astype(jnp.float32) < nb_act
    be_full = jnp.where(in_range, be, last_e)
    act = jnp.where(in_range & (rel < cnt_at), 1.0, 0.0)
    meta_ref[...] = jnp.concatenate([be_full, act], axis=0).astype(jnp.int32)


def _dispatch_body(x_hbm, pos0_hbm, pos1_hbm, xs_hbm,
                   idx0_v, idx1_v, rows_v, sem):
    wid = lax.axis_index("s") * 2 + lax.axis_index("c")
    tbase = wid * TOK_PER_TILE
    for c in range(TOK_PER_TILE // 32):
        tb = tbase + 32 * c
        pltpu.sync_copy(pos0_hbm.at[pl.ds(tb, 32)], idx0_v)
        pltpu.sync_copy(pos1_hbm.at[pl.ds(tb, 32)], idx1_v)
        pltpu.sync_copy(x_hbm.at[pl.ds(tb, 32)], rows_v)
        pltpu.sync_copy(rows_v, xs_hbm.at[idx0_v])
        pltpu.sync_copy(rows_v, xs_hbm.at[idx1_v])


def _make_dispatch():
    mesh = plsc.VectorSubcoreMesh(core_axis_name="c", subcore_axis_name="s")
    return functools.partial(
        pl.kernel, _dispatch_body, mesh=mesh,
        out_type=jax.ShapeDtypeStruct((CAP, HIDDEN), jnp.float32),
        scratch_types=[
            pltpu.VMEM((32,), jnp.int32),
            pltpu.VMEM((32,), jnp.int32),
            pltpu.VMEM((32, HIDDEN), jnp.float32),
            pltpu.SemaphoreType.DMA,
        ],
    )()


BI = INTER // 2


def _ffn_body(meta_ref, xs_ref, w1_ref, w2_ref, w3_ref, ys_ref):
    b = pl.program_id(0)
    ic = pl.program_id(1)

    @pl.when(meta_ref[1, b] > 0)
    def _():
        x = xs_ref[...]
        w1 = w1_ref[0]
        w3 = w3_ref[0]
        w2 = w2_ref[0]
        h = lax.dot_general(x, w1, (((1,), (1,)), ((), ())),
                            preferred_element_type=jnp.float32)
        u = lax.dot_general(x, w3, (((1,), (1,)), ((), ())),
                            preferred_element_type=jnp.float32)
        gg = h * jax.nn.sigmoid(h) * u
        o = lax.dot_general(gg, w2, (((1,), (1,)), ((), ())),
                            preferred_element_type=jnp.float32)

        @pl.when(ic == 0)
        def _():
            ys_ref[...] = o

        @pl.when(ic == 1)
        def _():
            ys_ref[...] += o


def _combine_body(ys_hbm, pos0_hbm, pos1_hbm, wts0_hbm, wts1_hbm, out_hbm,
                  idx0_v, idx1_v, w0_v, w1_v, bufa_v, bufb_v, outb_v,
                  sg0, sg1, sst):
    wid = lax.axis_index("s") * 2 + lax.axis_index("c")
    tb0 = wid * TOK_PER_TILE
    nch = TOK_PER_TILE // 16
    pltpu.sync_copy(pos0_hbm.at[pl.ds(wid * nch, nch)], idx0_v)
    pltpu.sync_copy(pos1_hbm.at[pl.ds(wid * nch, nch)], idx1_v)
    pltpu.sync_copy(wts0_hbm.at[pl.ds(tb0, TOK_PER_TILE)],
                    w0_v.at[pl.ds(0, TOK_PER_TILE)])
    pltpu.sync_copy(wts1_hbm.at[pl.ds(tb0, TOK_PER_TILE)],
                    w1_v.at[pl.ds(0, TOK_PER_TILE)])

    def issue(c):
        r = c % 2
        s = sg0 if r == 0 else sg1
        ca = pltpu.async_copy(ys_hbm.at[idx0_v.at[c]], bufa_v.at[r], s)
        cb = pltpu.async_copy(ys_hbm.at[idx1_v.at[c]], bufb_v.at[r], s)
        return ca, cb

    inflight = issue(0)
    store = None
    for c in range(nch):
        nxt = issue(c + 1) if c + 1 < nch else None
        inflight[0].wait()
        inflight[1].wait()
        inflight = nxt
        r = c % 2
        if store is not None:
            store.wait()

        def comb(j, carry):
            w0 = w0_v[pl.ds(c * 16 + j, 16)][0]
            w1 = w1_v[pl.ds(c * 16 + j, 16)][0]
            for cc in range(HIDDEN // 16):
                va = bufa_v[r, j, pl.ds(cc * 16, 16)]
                vb = bufb_v[r, j, pl.ds(cc * 16, 16)]
                outb_v[r, j, pl.ds(cc * 16, 16)] = w0 * va + w1 * vb
            return carry

        lax.fori_loop(0, 16, comb, 0)
        store = pltpu.async_copy(outb_v.at[r],
                                 out_hbm.at[pl.ds(tb0 + c * 16, 16)], sst)
    store.wait()


def _make_combine():
    mesh = plsc.VectorSubcoreMesh(core_axis_name="c", subcore_axis_name="s")
    return functools.partial(
        pl.kernel, _combine_body, mesh=mesh,
        out_type=jax.ShapeDtypeStruct((T, HIDDEN), jnp.float32),
        scratch_types=[
            pltpu.VMEM((TOK_PER_TILE // 16, 16), jnp.int32),
            pltpu.VMEM((TOK_PER_TILE // 16, 16), jnp.int32),
            pltpu.VMEM((TOK_PER_TILE + 16,), jnp.float32),
            pltpu.VMEM((TOK_PER_TILE + 16,), jnp.float32),
            pltpu.VMEM((2, 16, HIDDEN), jnp.float32),
            pltpu.VMEM((2, 16, HIDDEN), jnp.float32),
            pltpu.VMEM((2, 16, HIDDEN), jnp.float32),
            pltpu.SemaphoreType.DMA,
            pltpu.SemaphoreType.DMA,
            pltpu.SemaphoreType.DMA,
        ],
    )()


def _combine(ys, pos0, pos1, wts0, wts1):
    return _make_combine()(ys, pos0.reshape(T // 16, 16),
                           pos1.reshape(T // 16, 16), wts0, wts1)


def kernel(hidden_states, gate_w, w1, w2, w3):
    b, s, h = hidden_states.shape
    x = hidden_states.reshape(-1, h)

    pos0, pos1, wts0, wts1, meta = pl.pallas_call(
        _router_body,
        out_shape=[
            jax.ShapeDtypeStruct((T,), jnp.int32),
            jax.ShapeDtypeStruct((T,), jnp.int32),
            jax.ShapeDtypeStruct((T,), jnp.float32),
            jax.ShapeDtypeStruct((T,), jnp.float32),
            jax.ShapeDtypeStruct((2, 128), jnp.int32),
        ],
    )(x, gate_w)

    xs = _make_dispatch()(x, pos0, pos1)

    ys = pl.pallas_call(
        _ffn_body,
        grid_spec=pltpu.PrefetchScalarGridSpec(
            num_scalar_prefetch=1,
            grid=(NB, 2),
            in_specs=[
                pl.BlockSpec((BT, HIDDEN), lambda i, ic, m: (i, 0)),
                pl.BlockSpec((1, BI, HIDDEN),
                             lambda i, ic, m: (m[0, i],
                                               jnp.where(m[1, i] > 0, ic, 1), 0)),
                pl.BlockSpec((1, HIDDEN, BI),
                             lambda i, ic, m: (m[0, i], 0,
                                               jnp.where(m[1, i] > 0, ic, 1))),
                pl.BlockSpec((1, BI, HIDDEN),
                             lambda i, ic, m: (m[0, i],
                                               jnp.where(m[1, i] > 0, ic, 1), 0)),
            ],
            out_specs=pl.BlockSpec((BT, HIDDEN), lambda i, ic, m: (i, 0)),
        ),
        out_shape=jax.ShapeDtypeStruct((CAP, HIDDEN), jnp.float32),
        compiler_params=pltpu.CompilerParams(
            vmem_limit_bytes=62 * 1024 * 1024),
    )(meta, xs, w1, w2, w3)

    out = _combine(ys, pos0, pos1, wts0, wts1)
    return out.reshape(b, s, h)


# speedup vs baseline: 1.4544x; 1.0467x over previous
import functools
import jax
import jax.numpy as jnp
from jax import lax
from jax.experimental import pallas as pl
from jax.experimental.pallas import tpu as pltpu
from jax.experimental.pallas import tpu_sc as plsc

HIDDEN = 1024
INTER = 2048
NUM_EXPERTS = 8
T = 2048
BT = 512
CAP = T * 2 + NUM_EXPERTS * BT
NB = CAP // BT
NTILES = 32
SLOTS_PER_TILE = CAP // NTILES
TOK_PER_TILE = T // NTILES


def _router_body(x_ref, gate_ref, pos0_ref, pos1_ref, wts0_ref, wts1_ref,
                 meta_ref):
    x = x_ref[...]
    g = gate_ref[...]
    logits = lax.dot_general(x, g, (((1,), (1,)), ((), ())),
                             preferred_element_type=jnp.float32)
    p = jax.nn.softmax(logits, axis=1)
    eidx = lax.broadcasted_iota(jnp.int32, p.shape, 1)
    m1 = jnp.max(p, axis=1, keepdims=True)
    e0 = jnp.min(jnp.where(p == m1, eidx, NUM_EXPERTS), axis=1, keepdims=True)
    a0 = eidx == e0
    p2 = jnp.where(a0, -1.0, p)
    m2 = jnp.max(p2, axis=1, keepdims=True)
    e1 = jnp.min(jnp.where(p2 == m2, eidx, NUM_EXPERTS), axis=1, keepdims=True)
    a1 = eidx == e1
    denom = m1 + m2
    wts0_ref[...] = (m1 / denom).reshape(T)
    wts1_ref[...] = (m2 / denom).reshape(T)

    s = jnp.where(a0 | a1, 1.0, 0.0)
    r2 = lax.broadcasted_iota(jnp.int32, (T, T), 0)
    c2 = lax.broadcasted_iota(jnp.int32, (T, T), 1)
    ltri = jnp.where(c2 <= r2, 1.0, 0.0)
    cinc = lax.dot_general(ltri, s, (((1,), (0,)), ((), ())),
                           preferred_element_type=jnp.float32)
    cexc = cinc - s
    counts = cinc[T - 1:T, :]
    cnt_i = counts.astype(jnp.int32)
    pc_i = ((cnt_i + (BT - 1)) // BT) * BT
    pc_f = pc_i.astype(jnp.float32)
    er = lax.broadcasted_iota(jnp.int32, (NUM_EXPERTS, NUM_EXPERTS), 0)
    ec = lax.broadcasted_iota(jnp.int32, (NUM_EXPERTS, NUM_EXPERTS), 1)
    sut = jnp.where(er < ec, 1.0, 0.0)
    start_f = lax.dot_general(pc_f, sut, (((1,), (0,)), ((), ())),
                              preferred_element_type=jnp.float32)

    posmat = cexc + start_f
    pos0 = jnp.sum(jnp.where(a0, posmat, 0.0), axis=1, keepdims=True)
    pos1 = jnp.sum(jnp.where(a1, posmat, 0.0), axis=1, keepdims=True)
    pos0_ref[...] = pos0.astype(jnp.int32).reshape(T)
    pos1_ref[...] = pos1.astype(jnp.int32).reshape(T)

    bb = lax.broadcasted_iota(jnp.int32, (1, 128), 1)
    slot0 = (bb * BT).astype(jnp.float32)
    total_pad = jnp.sum(pc_f, axis=1, keepdims=True)
    nb_act = total_pad / float(BT)
    be = jnp.zeros((1, 128), jnp.float32)
    rel = jnp.zeros((1, 128), jnp.float32)
    cnt_at = jnp.zeros((1, 128), jnp.float32)
    for e in range(NUM_EXPERTS):
        st_e = start_f[0:1, e:e + 1]
        pc_e = pc_f[0:1, e:e + 1]
        ct_e = counts[0:1, e:e + 1]
        in_e = jnp.where((slot0 >= st_e) & (slot0 < st_e + pc_e), 1.0, 0.0)
        be = be + in_e * float(e)
        rel = rel + in_e * (slot0 - st_e)
        cnt_at = cnt_at + in_e * ct_e
    eids = lax.broadcasted_iota(jnp.int32, (1, NUM_EXPERTS), 1).astype(jnp.float32)
    last_e = jnp.max(jnp.where(pc_f > 0, eids, 0.0), axis=1, keepdims=True)
    in_range = bb.astype(jnp.float32) < nb_act
    be_full = jnp.where(in_range, be, last_e)
    act = jnp.where(in_range & (rel < cnt_at), 1.0, 0.0)
    nact = jnp.sum(act, axis=1, keepdims=True) + jnp.zeros((1, 128), jnp.float32)
    meta_ref[...] = jnp.concatenate([be_full, act, nact],
                                    axis=0).astype(jnp.int32)


def _dispatch_body(x_hbm, pos0_hbm, pos1_hbm, xs_hbm,
                   idx0_v, idx1_v, rows_v, sl0, sl1, ss0, ss1):
    wid = lax.axis_index("s") * 2 + lax.axis_index("c")
    tbase = wid * TOK_PER_TILE
    nch = TOK_PER_TILE // 32
    pltpu.sync_copy(pos0_hbm.at[pl.ds(wid * nch, nch)], idx0_v)
    pltpu.sync_copy(pos1_hbm.at[pl.ds(wid * nch, nch)], idx1_v)
    sls = [sl0, sl1]
    sss = [ss0, ss1]
    loads = [pltpu.async_copy(x_hbm.at[pl.ds(tbase + 32 * c, 32)],
                              rows_v.at[c % 2], sls[c % 2])
             for c in range(nch)]
    scats = []
    for c in range(nch):
        r = c % 2
        loads[c].wait()
        scats.append(pltpu.async_copy(rows_v.at[r], xs_hbm.at[idx0_v.at[c]],
                                      sss[r]))
        scats.append(pltpu.async_copy(rows_v.at[r], xs_hbm.at[idx1_v.at[c]],
                                      sss[r]))
    for s in scats:
        s.wait()


def _make_dispatch():
    mesh = plsc.VectorSubcoreMesh(core_axis_name="c", subcore_axis_name="s")
    return functools.partial(
        pl.kernel, _dispatch_body, mesh=mesh,
        out_type=jax.ShapeDtypeStruct((CAP, HIDDEN), jnp.float32),
        scratch_types=[
            pltpu.VMEM((TOK_PER_TILE // 32, 32), jnp.int32),
            pltpu.VMEM((TOK_PER_TILE // 32, 32), jnp.int32),
            pltpu.VMEM((2, 32, HIDDEN), jnp.float32),
            pltpu.SemaphoreType.DMA,
            pltpu.SemaphoreType.DMA,
            pltpu.SemaphoreType.DMA,
            pltpu.SemaphoreType.DMA,
        ],
    )()


BI = INTER // 2


def _ffn_body(meta_ref, xs_ref, w1_ref, w2_ref, w3_ref, ys_ref):
    b = pl.program_id(0)
    ic = pl.program_id(1)

    @pl.when(meta_ref[1, b] > 0)
    def _():
        x = xs_ref[...]
        w1 = w1_ref[0]
        w3 = w3_ref[0]
        w2 = w2_ref[0]
        h = lax.dot_general(x, w1, (((1,), (1,)), ((), ())),
                            preferred_element_type=jnp.float32)
        u = lax.dot_general(x, w3, (((1,), (1,)), ((), ())),
                            preferred_element_type=jnp.float32)
        gg = h * jax.nn.sigmoid(h) * u
        o = lax.dot_general(gg, w2, (((1,), (1,)), ((), ())),
                            preferred_element_type=jnp.float32)

        @pl.when(ic == 0)
        def _():
            ys_ref[...] = o

        @pl.when(ic == 1)
        def _():
            ys_ref[...] += o


def _combine_body(ys_hbm, pos0_hbm, pos1_hbm, wts0_hbm, wts1_hbm, out_hbm,
                  idx0_v, idx1_v, w0_v, w1_v, bufa_v, bufb_v, outb_v,
                  sg0, sg1, sst):
    wid = lax.axis_index("s") * 2 + lax.axis_index("c")
    tb0 = wid * TOK_PER_TILE
    nch = TOK_PER_TILE // 16
    pltpu.sync_copy(pos0_hbm.at[pl.ds(wid * nch, nch)], idx0_v)
    pltpu.sync_copy(pos1_hbm.at[pl.ds(wid * nch, nch)], idx1_v)
    pltpu.sync_copy(wts0_hbm.at[pl.ds(tb0, TOK_PER_TILE)],
                    w0_v.at[pl.ds(0, TOK_PER_TILE)])
    pltpu.sync_copy(wts1_hbm.at[pl.ds(tb0, TOK_PER_TILE)],
                    w1_v.at[pl.ds(0, TOK_PER_TILE)])

    def issue(c):
        r = c % 2
        s = sg0 if r == 0 else sg1
        ca = pltpu.async_copy(ys_hbm.at[idx0_v.at[c]], bufa_v.at[r], s)
        cb = pltpu.async_copy(ys_hbm.at[idx1_v.at[c]], bufb_v.at[r], s)
        return ca, cb

    inflight = issue(0)
    store = None
    for c in range(nch):
        nxt = issue(c + 1) if c + 1 < nch else None
        inflight[0].wait()
        inflight[1].wait()
        inflight = nxt
        r = c % 2
        if store is not None:
            store.wait()

        def comb(j, carry):
            w0 = w0_v[pl.ds(c * 16 + j, 16)][0]
            w1 = w1_v[pl.ds(c * 16 + j, 16)][0]
            for cc in range(HIDDEN // 16):
                va = bufa_v[r, j, pl.ds(cc * 16, 16)]
                vb = bufb_v[r, j, pl.ds(cc * 16, 16)]
                outb_v[r, j, pl.ds(cc * 16, 16)] = w0 * va + w1 * vb
            return carry

        lax.fori_loop(0, 16, comb, 0)
        store = pltpu.async_copy(outb_v.at[r],
                                 out_hbm.at[pl.ds(tb0 + c * 16, 16)], sst)
    store.wait()


def _make_combine():
    mesh = plsc.VectorSubcoreMesh(core_axis_name="c", subcore_axis_name="s")
    return functools.partial(
        pl.kernel, _combine_body, mesh=mesh,
        out_type=jax.ShapeDtypeStruct((T, HIDDEN), jnp.float32),
        scratch_types=[
            pltpu.VMEM((TOK_PER_TILE // 16, 16), jnp.int32),
            pltpu.VMEM((TOK_PER_TILE // 16, 16), jnp.int32),
            pltpu.VMEM((TOK_PER_TILE + 16,), jnp.float32),
            pltpu.VMEM((TOK_PER_TILE + 16,), jnp.float32),
            pltpu.VMEM((2, 16, HIDDEN), jnp.float32),
            pltpu.VMEM((2, 16, HIDDEN), jnp.float32),
            pltpu.VMEM((2, 16, HIDDEN), jnp.float32),
            pltpu.SemaphoreType.DMA,
            pltpu.SemaphoreType.DMA,
            pltpu.SemaphoreType.DMA,
        ],
    )()


def _combine(ys, pos0, pos1, wts0, wts1):
    return _make_combine()(ys, pos0.reshape(T // 16, 16),
                           pos1.reshape(T // 16, 16), wts0, wts1)


def kernel(hidden_states, gate_w, w1, w2, w3):
    b, s, h = hidden_states.shape
    x = hidden_states.reshape(-1, h)

    pos0, pos1, wts0, wts1, meta = pl.pallas_call(
        _router_body,
        out_shape=[
            jax.ShapeDtypeStruct((T,), jnp.int32),
            jax.ShapeDtypeStruct((T,), jnp.int32),
            jax.ShapeDtypeStruct((T,), jnp.float32),
            jax.ShapeDtypeStruct((T,), jnp.float32),
            jax.ShapeDtypeStruct((3, 128), jnp.int32),
        ],
    )(x, gate_w)

    xs = _make_dispatch()(x, pos0.reshape(T // 32, 32),
                          pos1.reshape(T // 32, 32))

    ys = pl.pallas_call(
        _ffn_body,
        grid_spec=pltpu.PrefetchScalarGridSpec(
            num_scalar_prefetch=1,
            grid=(NB, 2),
            in_specs=[
                pl.BlockSpec((BT, HIDDEN),
                             lambda i, ic, m: (jnp.minimum(i, m[2, 0] - 1), 0)),
                pl.BlockSpec((1, BI, HIDDEN),
                             lambda i, ic, m: (m[0, i],
                                               jnp.where(m[1, i] > 0, ic, 1), 0)),
                pl.BlockSpec((1, HIDDEN, BI),
                             lambda i, ic, m: (m[0, i], 0,
                                               jnp.where(m[1, i] > 0, ic, 1))),
                pl.BlockSpec((1, BI, HIDDEN),
                             lambda i, ic, m: (m[0, i],
                                               jnp.where(m[1, i] > 0, ic, 1), 0)),
            ],
            out_specs=pl.BlockSpec(
                (BT, HIDDEN), lambda i, ic, m: (jnp.minimum(i, m[2, 0] - 1), 0)),
        ),
        out_shape=jax.ShapeDtypeStruct((CAP, HIDDEN), jnp.float32),
        compiler_params=pltpu.CompilerParams(
            vmem_limit_bytes=62 * 1024 * 1024),
    )(meta, xs, w1, w2, w3)

    out = _combine(ys, pos0, pos1, wts0, wts1)
    return out.reshape(b, s, h)
